# baseline (speedup 1.0000x reference)
import math

import jax
import jax.numpy as jnp
from jax.experimental import pallas as pl
from jax.experimental.pallas import tpu as pltpu

_BS = 8
_NCROPS = 10
_T = 32
_F = 2048
_D = 512
_KNEAR = 3
_KABN = 3
_KTOP = 4
_NSAMP = 16


def _step_kernel(x_ref, wq_ref, wk_ref, wv_ref, wo_ref, w1_ref, b1_ref,
                 w2_ref, b2_ref, w3_ref, b3_ref,
                 topk_ref, feat_ref, scores_ref, selsc_ref,
                 dist_scr, idx_scr, wqkv_s, wo_s, w1_s, w2_s):
    i = pl.program_id(0)
    phase_a = i < _BS

    bf = jnp.bfloat16
    f32 = jnp.float32
    dot = lambda a, b: jnp.dot(a.astype(bf), b, preferred_element_type=f32)

    @pl.when(i == 0)
    def _():
        wqkv_s[:, :_D] = wq_ref[...].astype(bf)
        wqkv_s[:, _D:2 * _D] = wk_ref[...].astype(bf)
        wqkv_s[:, 2 * _D:] = wv_ref[...].astype(bf)
        wo_s[...] = wo_ref[...].astype(bf)
        w1_s[...] = w1_ref[...].astype(bf)
        w2_s[...] = w2_ref[...].astype(bf)

    x = x_ref[0].reshape(_NCROPS * _T, _F)
    xb = x.astype(bf)
    qkv = jnp.dot(xb, wqkv_s[...], preferred_element_type=f32)
    q = qkv[:, :_D]
    k = qkv[:, _D:2 * _D]
    v = qkv[:, 2 * _D:]
    scale = 1.0 / math.sqrt(float(_D))
    outs = []
    for c in range(_NCROPS):
        qc = q[c * _T:(c + 1) * _T]
        kc = k[c * _T:(c + 1) * _T]
        vc = v[c * _T:(c + 1) * _T]
        logits = dot(qc, kc.astype(bf).T) * scale
        p = jax.nn.softmax(logits, axis=-1)
        outs.append(dot(p, vc.astype(bf)))
    o = jnp.concatenate(outs, axis=0)
    feats = jax.nn.relu(dot(o, wo_s[...]) + x)

    s = jax.nn.relu(dot(feats, w1_s[...]) + b1_ref[...])
    s = jax.nn.relu(dot(s, w2_s[...]) + b2_ref[...])
    sb = s.astype(bf).astype(f32)
    w3b = w3_ref[...].astype(bf).astype(f32)
    logit = jnp.sum(sb * w3b, axis=1, keepdims=True) + b3_ref[...]
    sc = jax.nn.sigmoid(logit).reshape(_NCROPS, _T)
    scores_mean = jnp.mean(sc, axis=0)
    scores_ref[0, 0, :] = scores_mean

    mag = jnp.sqrt(jnp.sum(feats * feats, axis=1) + 1e-12).reshape(_NCROPS, _T)
    temp = jnp.where(phase_a, 6.0, 5.0)
    w = jax.nn.softmax(mag / temp, axis=1)
    dist3 = feats.reshape(_NCROPS, _T, _F) * w[:, :, None] * float(_T)
    dist_scr[...] = dist3

    cur = dist3.reshape(_NCROPS * _T, _F)
    neg = jnp.full((_NCROPS * _T, 128), -jnp.inf, jnp.float32)
    s0 = cur[:, :128]
    s1 = neg
    s2 = neg
    s3 = neg
    for j in range(1, _F // 128):
        c = cur[:, j * 128:(j + 1) * 128]
        t0 = jnp.maximum(s0, c)
        c = jnp.minimum(s0, c)
        s0 = t0
        t1 = jnp.maximum(s1, c)
        c = jnp.minimum(s1, c)
        s1 = t1
        t2 = jnp.maximum(s2, c)
        c = jnp.minimum(s2, c)
        s2 = t2
        s3 = jnp.maximum(s3, c)
    cand = jnp.concatenate([s0, s1, s2, s3], axis=1)
    lane = jax.lax.broadcasted_iota(jnp.int32, cand.shape, 1)
    vals = []
    for j in range(_KTOP):
        m = jnp.max(cand, axis=1, keepdims=True)
        vals.append(m)
        if j < _KTOP - 1:
            first = jnp.min(jnp.where(cand == m, lane, _F), axis=1,
                            keepdims=True)
            cand = jnp.where(lane == first, -jnp.inf, cand)
    topk_ref[0] = jnp.concatenate(vals, axis=1).reshape(_NCROPS, _T, _KTOP)

    @pl.when(phase_a)
    def _():
        feat2 = jnp.mean(feats.reshape(_NCROPS, _T, _F), axis=0)
        ad = jnp.abs(feat2[_KNEAR:, :] - feat2[:_T - _KNEAR, :])
        diff = jnp.mean(ad, axis=1).reshape(1, _T - _KNEAR)
        lane29 = jax.lax.broadcasted_iota(jnp.int32, (1, _T - _KNEAR), 1)
        c2 = diff
        for j in range(_KABN):
            m = jnp.max(c2)
            first = jnp.min(jnp.where(c2 == m, lane29, _T))
            idx_scr[i, j] = first + _KNEAR
            c2 = jnp.where(lane29 == first, -jnp.inf, c2)

    row = jnp.where(phase_a, i, i - _BS)
    lane32 = jax.lax.broadcasted_iota(jnp.int32, (1, _T), 1)
    sm = scores_mean.reshape(1, _T)
    acc = jnp.zeros((), jnp.float32)
    for kk in range(_KABN):
        jk = idx_scr[row, kk]
        feat_ref[:, 0, kk, :] = dist_scr[:, pl.ds(jk, 1), :].reshape(
            _NCROPS, _F)
        acc = acc + jnp.sum(jnp.where(lane32 == jk, sm, 0.0))
    selsc_ref[...] = jnp.broadcast_to((acc / float(_KABN)).reshape(1, 1),
                                      (1, 1, 128))


def _smap(i):
    return jnp.where(i < _BS, i + _BS, i - _BS)


def kernel(inputs, Wq, Wk, Wv, Wo, W1, b1, W2, b2, W3, b3):
    const2 = lambda i: (0, 0)
    in_specs = [
        pl.BlockSpec((1, _NCROPS, _T, _F), lambda i: (_smap(i), 0, 0, 0)),
        pl.BlockSpec((_F, _D), const2),
        pl.BlockSpec((_F, _D), const2),
        pl.BlockSpec((_F, _D), const2),
        pl.BlockSpec((_D, _F), const2),
        pl.BlockSpec((_F, _D), const2),
        pl.BlockSpec((1, _D), const2),
        pl.BlockSpec((_D, 128), const2),
        pl.BlockSpec((1, 128), const2),
        pl.BlockSpec((1, 128), const2),
        pl.BlockSpec((1, 1), const2),
    ]
    out_specs = [
        pl.BlockSpec((1, _NCROPS, _T, _KTOP), lambda i: (_smap(i), 0, 0, 0)),
        pl.BlockSpec((_NCROPS, 1, _KABN, _F), lambda i: (0, _smap(i), 0, 0)),
        pl.BlockSpec((1, 1, _T), lambda i: (_smap(i), 0, 0)),
        pl.BlockSpec((1, 1, 128), lambda i: (_smap(i), 0, 0)),
    ]
    out_shapes = [
        jax.ShapeDtypeStruct((_NSAMP, _NCROPS, _T, _KTOP), jnp.float32),
        jax.ShapeDtypeStruct((_NCROPS, _NSAMP, _KABN, _F), jnp.float32),
        jax.ShapeDtypeStruct((_NSAMP, 1, _T), jnp.float32),
        jax.ShapeDtypeStruct((_NSAMP, 1, 128), jnp.float32),
    ]
    topk_all, feat_all, scores_all, selsc_all = pl.pallas_call(
        _step_kernel,
        grid=(_NSAMP,),
        in_specs=in_specs,
        out_specs=out_specs,
        out_shape=out_shapes,
        scratch_shapes=[
            pltpu.VMEM((_NCROPS, _T, _F), jnp.float32),
            pltpu.SMEM((_BS, _KABN), jnp.int32),
            pltpu.VMEM((_F, 3 * _D), jnp.bfloat16),
            pltpu.VMEM((_D, _F), jnp.bfloat16),
            pltpu.VMEM((_F, _D), jnp.bfloat16),
            pltpu.VMEM((_D, 128), jnp.bfloat16),
        ],
    )(inputs, Wq, Wk, Wv, Wo, W1, b1.reshape(1, _D),
      W2, b2.reshape(1, 128), W3.reshape(1, 128), b3.reshape(1, 1))

    topk_n_vals = topk_all[:_BS].reshape(_BS * _NCROPS, _T, _KTOP)
    topk_ab_vals = topk_all[_BS:].reshape(_BS * _NCROPS, _T, _KTOP)
    feat_normal = feat_all[:, :_BS].reshape(_NCROPS * _BS, _KABN, _F)
    feat_abnormal = feat_all[:, _BS:].reshape(_NCROPS * _BS, _KABN, _F)
    scores_out = scores_all.reshape(_NSAMP, _T, 1)
    score_normal = selsc_all[:_BS, 0, :1]
    score_abnormal = selsc_all[_BS:, 0, :1]
    return (score_abnormal, score_normal, topk_ab_vals, topk_n_vals,
            feat_abnormal, feat_normal, scores_out)


# speedup vs baseline: 1.0574x; 1.0574x over previous
import math

import jax
import jax.numpy as jnp
from jax.experimental import pallas as pl
from jax.experimental.pallas import tpu as pltpu

_BS = 8
_NCROPS = 10
_T = 32
_F = 2048
_D = 512
_KNEAR = 3
_KABN = 3
_KTOP = 4
_NSAMP = 16
_PAIR = 2
_NP = _NSAMP // _PAIR
_NPH = _NP // 2
_R = _PAIR * _NCROPS * _T


def _step_kernel(x_ref, wq_ref, wk_ref, wv_ref, wo_ref, w1_ref, b1_ref,
                 w2_ref, b2_ref, w3_ref, b3_ref,
                 topk_ref, feat_ref, scores_ref, selsc_ref,
                 dist_scr, idx_scr, wqkv_s, wo_s, w1_s, w2_s):
    i = pl.program_id(0)
    phase_a = i < _NPH

    bf = jnp.bfloat16
    f32 = jnp.float32
    dot = lambda a, b: jnp.dot(a.astype(bf), b, preferred_element_type=f32)

    @pl.when(i == 0)
    def _():
        wqkv_s[:, :_D] = wq_ref[...].astype(bf)
        wqkv_s[:, _D:2 * _D] = wk_ref[...].astype(bf)
        wqkv_s[:, 2 * _D:] = wv_ref[...].astype(bf)
        wo_s[...] = wo_ref[...].astype(bf)
        w1_s[...] = w1_ref[...].astype(bf)
        w2_s[...] = w2_ref[...].astype(bf)

    x = x_ref[...].reshape(_R, _F)
    xb = x.astype(bf)
    qkv = jnp.dot(xb, wqkv_s[...], preferred_element_type=f32)
    q = qkv[:, :_D]
    k = qkv[:, _D:2 * _D]
    v = qkv[:, 2 * _D:]
    scale = 1.0 / math.sqrt(float(_D))
    outs = []
    for c in range(_PAIR * _NCROPS):
        qc = q[c * _T:(c + 1) * _T]
        kc = k[c * _T:(c + 1) * _T]
        vc = v[c * _T:(c + 1) * _T]
        logits = dot(qc, kc.astype(bf).T) * scale
        p = jax.nn.softmax(logits, axis=-1)
        outs.append(dot(p, vc.astype(bf)))
    o = jnp.concatenate(outs, axis=0)
    feats = jax.nn.relu(dot(o, wo_s[...]) + x)

    s = jax.nn.relu(dot(feats, w1_s[...]) + b1_ref[...])
    s = jax.nn.relu(dot(s, w2_s[...]) + b2_ref[...])
    sb = s.astype(bf).astype(f32)
    w3b = w3_ref[...].astype(bf).astype(f32)
    logit = jnp.sum(sb * w3b, axis=1, keepdims=True) + b3_ref[...]
    sc = jax.nn.sigmoid(logit).reshape(_PAIR, _NCROPS, _T)
    sm2 = jnp.mean(sc, axis=1)
    scores_ref[0] = sm2

    mag = jnp.sqrt(jnp.sum(feats * feats, axis=1)
                   + 1e-12).reshape(_PAIR * _NCROPS, _T)
    temp = jnp.where(phase_a, 6.0, 5.0)
    w = jax.nn.softmax(mag / temp, axis=1)
    dist3 = feats.reshape(_PAIR * _NCROPS, _T, _F) * w[:, :, None] * float(_T)
    dist_scr[...] = dist3

    cur = dist3.reshape(_R, _F)
    neg = jnp.full((_R, 128), -jnp.inf, jnp.float32)
    s0 = cur[:, :128]
    s1 = neg
    s2 = neg
    s3 = neg
    for j in range(1, _F // 128):
        c = cur[:, j * 128:(j + 1) * 128]
        t0 = jnp.maximum(s0, c)
        c = jnp.minimum(s0, c)
        s0 = t0
        t1 = jnp.maximum(s1, c)
        c = jnp.minimum(s1, c)
        s1 = t1
        t2 = jnp.maximum(s2, c)
        c = jnp.minimum(s2, c)
        s2 = t2
        s3 = jnp.maximum(s3, c)
    cand = jnp.concatenate([s0, s1, s2, s3], axis=1)
    lane = jax.lax.broadcasted_iota(jnp.int32, cand.shape, 1)
    vals = []
    for j in range(_KTOP):
        m = jnp.max(cand, axis=1, keepdims=True)
        vals.append(m)
        if j < _KTOP - 1:
            first = jnp.min(jnp.where(cand == m, lane, _F), axis=1,
                            keepdims=True)
            cand = jnp.where(lane == first, -jnp.inf, cand)
    topk_ref[0] = jnp.concatenate(vals, axis=1).reshape(
        _PAIR, _NCROPS, _T, _KTOP)

    feat2 = jnp.mean(feats.reshape(_PAIR, _NCROPS, _T, _F), axis=1)

    @pl.when(phase_a)
    def _():
        lane29 = jax.lax.broadcasted_iota(jnp.int32, (1, _T - _KNEAR), 1)
        for sidx in range(_PAIR):
            f2 = feat2[sidx]
            ad = jnp.abs(f2[_KNEAR:, :] - f2[:_T - _KNEAR, :])
            c2 = jnp.mean(ad, axis=1).reshape(1, _T - _KNEAR)
            for j in range(_KABN):
                m = jnp.max(c2)
                first = jnp.min(jnp.where(c2 == m, lane29, _T))
                idx_scr[_PAIR * i + sidx, j] = first + _KNEAR
                c2 = jnp.where(lane29 == first, -jnp.inf, c2)

    prow = jnp.where(phase_a, i, i - _NPH)
    lane32 = jax.lax.broadcasted_iota(jnp.int32, (1, _T), 1)
    sels = []
    for sidx in range(_PAIR):
        smr = sm2[sidx].reshape(1, _T)
        acc = jnp.zeros((), jnp.float32)
        for kk in range(_KABN):
            jk = idx_scr[_PAIR * prow + sidx, kk]
            feat_ref[0, sidx, kk] = dist_scr[
                sidx * _NCROPS:(sidx + 1) * _NCROPS,
                pl.ds(jk, 1), :].reshape(_NCROPS, _F)
            acc = acc + jnp.sum(jnp.where(lane32 == jk, smr, 0.0))
        sels.append(jnp.broadcast_to((acc / float(_KABN)).reshape(1, 1),
                                     (1, 128)))
    selsc_ref[0] = jnp.concatenate(sels, axis=0)


def _pmap(i):
    return jnp.where(i < _NPH, i + _NPH, i - _NPH)


def kernel(inputs, Wq, Wk, Wv, Wo, W1, b1, W2, b2, W3, b3):
    const2 = lambda i: (0, 0)
    in_specs = [
        pl.BlockSpec((_PAIR, _NCROPS, _T, _F),
                     lambda i: (_pmap(i), 0, 0, 0)),
        pl.BlockSpec((_F, _D), const2),
        pl.BlockSpec((_F, _D), const2),
        pl.BlockSpec((_F, _D), const2),
        pl.BlockSpec((_D, _F), const2),
        pl.BlockSpec((_F, _D), const2),
        pl.BlockSpec((1, _D), const2),
        pl.BlockSpec((_D, 128), const2),
        pl.BlockSpec((1, 128), const2),
        pl.BlockSpec((1, 128), const2),
        pl.BlockSpec((1, 1), const2),
    ]
    out_specs = [
        pl.BlockSpec((1, _PAIR, _NCROPS, _T, _KTOP),
                     lambda i: (_pmap(i), 0, 0, 0, 0)),
        pl.BlockSpec((1, _PAIR, _KABN, _NCROPS, _F),
                     lambda i: (_pmap(i), 0, 0, 0, 0)),
        pl.BlockSpec((1, _PAIR, _T), lambda i: (_pmap(i), 0, 0)),
        pl.BlockSpec((1, _PAIR, 128), lambda i: (_pmap(i), 0, 0)),
    ]
    out_shapes = [
        jax.ShapeDtypeStruct((_NP, _PAIR, _NCROPS, _T, _KTOP), jnp.float32),
        jax.ShapeDtypeStruct((_NP, _PAIR, _KABN, _NCROPS, _F), jnp.float32),
        jax.ShapeDtypeStruct((_NP, _PAIR, _T), jnp.float32),
        jax.ShapeDtypeStruct((_NP, _PAIR, 128), jnp.float32),
    ]
    topk_all, feat_all, scores_all, selsc_all = pl.pallas_call(
        _step_kernel,
        grid=(_NP,),
        in_specs=in_specs,
        out_specs=out_specs,
        out_shape=out_shapes,
        scratch_shapes=[
            pltpu.VMEM((_PAIR * _NCROPS, _T, _F), jnp.float32),
            pltpu.SMEM((_BS, _KABN), jnp.int32),
            pltpu.VMEM((_F, 3 * _D), jnp.bfloat16),
            pltpu.VMEM((_D, _F), jnp.bfloat16),
            pltpu.VMEM((_F, _D), jnp.bfloat16),
            pltpu.VMEM((_D, 128), jnp.bfloat16),
        ],
    )(inputs, Wq, Wk, Wv, Wo, W1, b1.reshape(1, _D),
      W2, b2.reshape(1, 128), W3.reshape(1, 128), b3.reshape(1, 1))

    topk_s = topk_all.reshape(_NSAMP, _NCROPS, _T, _KTOP)
    topk_n_vals = topk_s[:_BS].reshape(_BS * _NCROPS, _T, _KTOP)
    topk_ab_vals = topk_s[_BS:].reshape(_BS * _NCROPS, _T, _KTOP)
    feat_s = feat_all.reshape(_NSAMP, _KABN, _NCROPS, _F)
    feat_normal = feat_s[:_BS].transpose(2, 0, 1, 3).reshape(
        _NCROPS * _BS, _KABN, _F)
    feat_abnormal = feat_s[_BS:].transpose(2, 0, 1, 3).reshape(
        _NCROPS * _BS, _KABN, _F)
    scores_out = scores_all.reshape(_NSAMP, _T, 1)
    selsc_s = selsc_all.reshape(_NSAMP, 128)
    score_normal = selsc_s[:_BS, :1]
    score_abnormal = selsc_s[_BS:, :1]
    return (score_abnormal, score_normal, topk_ab_vals, topk_n_vals,
            feat_abnormal, feat_normal, scores_out)
